# baseline (speedup 1.0000x reference)
import functools

import jax
import jax.numpy as jnp
from jax.experimental import pallas as pl


def _gcn_block(x_ref, t_ref, w_ref, b_ref, o_ref, *, k_order: int):
    x = x_ref[0]
    acc = None
    for k in range(k_order):
        temp = jnp.dot(t_ref[k], x, preferred_element_type=jnp.float32)
        part = jnp.dot(temp, w_ref[k], preferred_element_type=jnp.float32)
        acc = part if acc is None else acc + part
    o_ref[0] = acc + b_ref[...]


@jax.jit
def kernel(input, T_k, weight, bias):
    B, V, D_IN = input.shape
    K, _, D_OUT = weight.shape
    TILE_V = min(256, V)

    bias2d = bias.reshape(1, D_OUT)

    out = pl.pallas_call(
        functools.partial(_gcn_block, k_order=K),
        grid=(B, V // TILE_V),
        in_specs=[
            pl.BlockSpec((1, V, D_IN), lambda b, i: (b, 0, 0)),
            pl.BlockSpec((K, TILE_V, V), lambda b, i: (0, i, 0)),
            pl.BlockSpec((K, D_IN, D_OUT), lambda b, i: (0, 0, 0)),
            pl.BlockSpec((1, D_OUT), lambda b, i: (0, 0)),
        ],
        out_specs=pl.BlockSpec((1, TILE_V, D_OUT), lambda b, i: (b, i, 0)),
        out_shape=jax.ShapeDtypeStruct((B, V, D_OUT), jnp.float32),
    )(input, T_k, weight, bias2d)
    return out


# speedup vs baseline: 1.1031x; 1.1031x over previous
import jax
import jax.numpy as jnp
from jax.experimental import pallas as pl


def _gcn_block(x_ref, t_ref, w_ref, b_ref, o_ref):
    k = pl.program_id(1)

    @pl.when(k == 0)
    def _init():
        o_ref[...] = jnp.broadcast_to(b_ref[...], o_ref.shape)

    t = t_ref[0].astype(jnp.bfloat16)
    w = w_ref[0]
    n_batch = x_ref.shape[0]
    for b in range(n_batch):
        temp = jnp.dot(t, x_ref[b], preferred_element_type=jnp.float32)
        part = jnp.dot(temp.astype(jnp.bfloat16), w,
                       preferred_element_type=jnp.float32)
        o_ref[b] = o_ref[b] + part


@jax.jit
def kernel(input, T_k, weight, bias):
    B, V, D_IN = input.shape
    K, _, D_OUT = weight.shape
    TILE_V = min(512, V)

    x16 = input.astype(jnp.bfloat16)
    w16 = weight.astype(jnp.bfloat16)
    bias2d = bias.reshape(1, D_OUT)

    out = pl.pallas_call(
        _gcn_block,
        grid=(V // TILE_V, K),
        in_specs=[
            pl.BlockSpec((B, V, D_IN), lambda i, k: (0, 0, 0)),
            pl.BlockSpec((1, TILE_V, V), lambda i, k: (k, i, 0)),
            pl.BlockSpec((1, D_IN, D_OUT), lambda i, k: (k, 0, 0)),
            pl.BlockSpec((1, D_OUT), lambda i, k: (0, 0)),
        ],
        out_specs=pl.BlockSpec((B, TILE_V, D_OUT), lambda i, k: (0, i, 0)),
        out_shape=jax.ShapeDtypeStruct((B, V, D_OUT), jnp.float32),
    )(x16, T_k, w16, bias2d)
    return out
